# baseline (speedup 1.0000x reference)
import dataclasses
import functools

import jax
import jax.numpy as jnp
from jax import lax
from jax.experimental import pallas as pl
from jax.experimental.pallas import tpu as pltpu
from jax.experimental.pallas import tpu_sc as plsc

_NW = 32
_L = 16


def _tc_nll_body(SB, B, V, x_ref, t_ref, m_ref, out_ref, tT_ref, mT_ref):
    i = pl.program_id(0)

    @pl.when(i == 0)
    def _():
        out_ref[...] = jnp.zeros_like(out_ref)
        tT_ref[...] = t_ref[...].T
        mT_ref[...] = m_ref[...].T

    x = x_ref[...]
    t = tT_ref[pl.ds(i * SB, SB), :]
    m = mT_ref[pl.ds(i * SB, SB), :]
    col = lax.broadcasted_iota(jnp.int32, (SB, B, V), 2)
    sel = jnp.where(col == t[:, :, None], x, 0.0)
    nll_blk = jnp.sum(jnp.sum(sel, axis=2) * m)
    msk_blk = jnp.sum(m)
    r8 = lax.broadcasted_iota(jnp.int32, (8, 128), 0)
    c128 = lax.broadcasted_iota(jnp.int32, (8, 128), 1)
    out_ref[...] += jnp.where((r8 == 0) & (c128 == 0), nll_blk, 0.0) \
        + jnp.where((r8 == 0) & (c128 == 1), msk_blk, 0.0)


def _sc_match_body(N, MW, G, CH2, CHR,
                   mif_hbm, mtf_hbm, mmf_hbm, out_hbm,
                   mt_v, mm_v, idx2_v, val2_v, acc_v, cnt_v, res_v, sem):
    wid = lax.axis_index("s") * 2 + lax.axis_index("c")
    iota = lax.iota(jnp.int32, _L)
    zero = jnp.zeros((_L,), jnp.float32)

    b2 = wid * CH2
    pltpu.sync_copy(mtf_hbm.at[pl.ds(b2, CH2)], mt_v)
    pltpu.sync_copy(mmf_hbm.at[pl.ds(b2, CH2)], mm_v.at[pl.ds(0, CH2)])
    for c in range(CH2 // _L, (G * CHR) // _L):
        mm_v[pl.ds(c * _L, _L)] = zero

    @pl.loop(0, CH2, step=_L)
    def _(c):
        j = b2 + c + iota
        row = jnp.minimum(lax.shift_right_logical(j, 2), N - 1)
        mt = mt_v[pl.ds(c, _L)]
        idx2_v[pl.ds(c, _L)] = jnp.maximum((mt - 1) * N + row, 0)

    splits = list(range(0, CH2, 128)) + [CH2]
    cps = []
    for lo, hi in zip(splits[:-1], splits[1:]):
        cps.append(pltpu.async_copy(
            mif_hbm.at[idx2_v.at[pl.ds(lo, hi - lo)]],
            val2_v.at[pl.ds(lo, hi - lo)], sem))
    for cp in cps:
        cp.wait()

    acc_v[...] = zero
    cnt_v[...] = zero

    @pl.loop(0, CH2, step=_L)
    def _(c):
        mt = mt_v[pl.ds(c, _L)]
        mm = mm_v[pl.ds(c, _L)]
        term = val2_v[pl.ds(c, _L)] * jnp.where(mt == 0, 0.0, mm)
        acc_v[...] = acc_v[...] + term

    @pl.loop(0, CHR * G, step=_L * G)
    def _(base):
        rs = zero
        for g in range(G):
            rs = rs + plsc.load_gather(mm_v, [iota * G + (base + g)])
        cnt_v[...] = cnt_v[...] + jnp.where(rs != 0.0, 1.0, 0.0)

    s2 = jnp.sum(acc_v[...])
    sc = jnp.sum(cnt_v[...])
    res_v[...] = jnp.where(iota == 0, -s2, 0.0) + jnp.where(iota == 1, sc, 0.0)
    pltpu.sync_copy(res_v, out_hbm.at[wid])


def kernel(input, target, mask, match_input, match_target, match_mask):
    B, S, V = input.shape
    MW = match_input.shape[2]
    G = match_target.shape[2]
    N = B * S

    f32 = jnp.float32
    i32 = jnp.int32
    xt = jnp.transpose(input, (1, 0, 2))
    SB = 5

    nll = pl.pallas_call(
        functools.partial(_tc_nll_body, SB, B, V),
        grid=(S // SB,),
        in_specs=[
            pl.BlockSpec((SB, B, V), lambda i: (i, 0, 0)),
            pl.BlockSpec((B, S), lambda i: (0, 0)),
            pl.BlockSpec((B, S), lambda i: (0, 0)),
        ],
        out_specs=pl.BlockSpec((8, 128), lambda i: (0, 0)),
        out_shape=jax.ShapeDtypeStruct((8, 128), f32),
        scratch_shapes=[
            pltpu.VMEM((S, B), i32),
            pltpu.VMEM((S, B), f32),
        ],
    )(xt, target.astype(i32), mask.astype(f32))

    CH2 = N * G // _NW
    CHR = N // _NW
    CHR_PAD = (CHR + _L - 1) // _L * _L

    mif = jnp.transpose(match_input, (2, 0, 1)).reshape(-1)
    mtf = match_target.reshape(-1).astype(i32)
    mmf = match_mask.reshape(-1).astype(f32)

    mesh = plsc.VectorSubcoreMesh(core_axis_name="c", subcore_axis_name="s")
    body = functools.partial(_sc_match_body, N, MW, G, CH2, CHR_PAD)
    cp = pltpu.CompilerParams()
    if "needs_layout_passes" in pltpu.CompilerParams.__dataclass_fields__:
        cp = dataclasses.replace(cp, needs_layout_passes=False)
    out = pl.kernel(
        body,
        out_type=jax.ShapeDtypeStruct((_NW, _L), f32),
        mesh=mesh,
        compiler_params=cp,
        scratch_types=[
            pltpu.VMEM((CH2,), i32),
            pltpu.VMEM((G * CHR_PAD,), f32),
            pltpu.VMEM((CH2,), i32),
            pltpu.VMEM((CH2,), f32),
            pltpu.VMEM((_L,), f32),
            pltpu.VMEM((_L,), f32),
            pltpu.VMEM((_L,), f32),
            pltpu.SemaphoreType.DMA,
        ],
    )(mif, mtf, mmf)

    p = out.sum(axis=0)
    return (-nll[0, 0] / nll[0, 1], p[0] / p[1])


# speedup vs baseline: 1.0365x; 1.0365x over previous
import dataclasses
import functools

import jax
import jax.numpy as jnp
from jax import lax
from jax.experimental import pallas as pl
from jax.experimental.pallas import tpu as pltpu
from jax.experimental.pallas import tpu_sc as plsc

_NW = 16
_L = 16


def _tc_nll_body(SB, B, V, x_ref, t_ref, m_ref, out_ref, tT_ref, mT_ref):
    i = pl.program_id(0)

    @pl.when(i == 0)
    def _():
        out_ref[...] = jnp.zeros_like(out_ref)
        tT_ref[...] = t_ref[...].T
        mT_ref[...] = m_ref[...].T

    x = x_ref[...]
    t = tT_ref[pl.ds(i * SB, SB), :]
    m = mT_ref[pl.ds(i * SB, SB), :]
    col = lax.broadcasted_iota(jnp.int32, (SB, B, V), 2)
    sel = jnp.where(col == t[:, :, None], x, 0.0)
    nll_blk = jnp.sum(jnp.sum(sel, axis=2) * m)
    msk_blk = jnp.sum(m)
    r8 = lax.broadcasted_iota(jnp.int32, (8, 128), 0)
    c128 = lax.broadcasted_iota(jnp.int32, (8, 128), 1)
    out_ref[...] += jnp.where((r8 == 0) & (c128 == 0), nll_blk, 0.0) \
        + jnp.where((r8 == 0) & (c128 == 1), msk_blk, 0.0)


def _sc_match_body(N, MW, G, CH2, CHR,
                   mif_hbm, mtf_hbm, mmf_hbm, out_hbm,
                   mt_v, mm_v, idx2_v, val2_v, acc_v, cnt_v, res_v, sem):
    wid = lax.axis_index("s") + lax.axis_index("c") * 16
    iota = lax.iota(jnp.int32, _L)
    zero = jnp.zeros((_L,), jnp.float32)

    b2 = wid * CH2
    pltpu.sync_copy(mtf_hbm.at[pl.ds(b2, CH2)], mt_v)
    pltpu.sync_copy(mmf_hbm.at[pl.ds(b2, CH2)], mm_v.at[pl.ds(0, CH2)])
    for c in range(CH2 // _L, (G * CHR) // _L):
        mm_v[pl.ds(c * _L, _L)] = zero

    @pl.loop(0, CH2, step=_L)
    def _(c):
        j = b2 + c + iota
        row = jnp.minimum(lax.shift_right_logical(j, 2), N - 1)
        mt = mt_v[pl.ds(c, _L)]
        idx2_v[pl.ds(c, _L)] = jnp.maximum((mt - 1) * N + row, 0)

    splits = list(range(0, CH2, 128)) + [CH2]
    cps = []
    for lo, hi in zip(splits[:-1], splits[1:]):
        cps.append(pltpu.async_copy(
            mif_hbm.at[idx2_v.at[pl.ds(lo, hi - lo)]],
            val2_v.at[pl.ds(lo, hi - lo)], sem))
    for cp in cps:
        cp.wait()

    acc_v[...] = zero
    cnt_v[...] = zero

    @pl.loop(0, CH2, step=_L)
    def _(c):
        mt = mt_v[pl.ds(c, _L)]
        mm = mm_v[pl.ds(c, _L)]
        term = val2_v[pl.ds(c, _L)] * jnp.where(mt == 0, 0.0, mm)
        acc_v[...] = acc_v[...] + term

    @pl.loop(0, CHR * G, step=_L * G)
    def _(base):
        rs = zero
        for g in range(G):
            rs = rs + plsc.load_gather(mm_v, [iota * G + (base + g)])
        cnt_v[...] = cnt_v[...] + jnp.where(rs != 0.0, 1.0, 0.0)

    s2 = jnp.sum(acc_v[...])
    sc = jnp.sum(cnt_v[...])
    res_v[...] = jnp.where(iota == 0, -s2, 0.0) + jnp.where(iota == 1, sc, 0.0)
    pltpu.sync_copy(res_v, out_hbm.at[wid])


def kernel(input, target, mask, match_input, match_target, match_mask):
    B, S, V = input.shape
    MW = match_input.shape[2]
    G = match_target.shape[2]
    N = B * S

    f32 = jnp.float32
    i32 = jnp.int32
    xt = jnp.transpose(input, (1, 0, 2))
    SB = 10

    nll = pl.pallas_call(
        functools.partial(_tc_nll_body, SB, B, V),
        grid=(S // SB,),
        in_specs=[
            pl.BlockSpec((SB, B, V), lambda i: (i, 0, 0)),
            pl.BlockSpec((B, S), lambda i: (0, 0)),
            pl.BlockSpec((B, S), lambda i: (0, 0)),
        ],
        out_specs=pl.BlockSpec((8, 128), lambda i: (0, 0)),
        out_shape=jax.ShapeDtypeStruct((8, 128), f32),
        scratch_shapes=[
            pltpu.VMEM((S, B), i32),
            pltpu.VMEM((S, B), f32),
        ],
    )(xt, target.astype(i32), mask.astype(f32))

    CH2 = N * G // _NW
    CHR = N // _NW
    CHR_PAD = (CHR + _L - 1) // _L * _L

    mif = jnp.transpose(match_input, (2, 0, 1)).reshape(-1)
    mtf = match_target.reshape(-1).astype(i32)
    mmf = match_mask.reshape(-1).astype(f32)

    mesh = plsc.VectorSubcoreMesh(core_axis_name="c", subcore_axis_name="s", num_cores=1)
    body = functools.partial(_sc_match_body, N, MW, G, CH2, CHR_PAD)
    cp = pltpu.CompilerParams()
    if "needs_layout_passes" in pltpu.CompilerParams.__dataclass_fields__:
        cp = dataclasses.replace(cp, needs_layout_passes=False)
    out = pl.kernel(
        body,
        out_type=jax.ShapeDtypeStruct((_NW, _L), f32),
        mesh=mesh,
        compiler_params=cp,
        scratch_types=[
            pltpu.VMEM((CH2,), i32),
            pltpu.VMEM((G * CHR_PAD,), f32),
            pltpu.VMEM((CH2,), i32),
            pltpu.VMEM((CH2,), f32),
            pltpu.VMEM((_L,), f32),
            pltpu.VMEM((_L,), f32),
            pltpu.VMEM((_L,), f32),
            pltpu.SemaphoreType.DMA,
        ],
    )(mif, mtf, mmf)

    p = out.sum(axis=0)
    return (-nll[0, 0] / nll[0, 1], p[0] / p[1])


# speedup vs baseline: 1.0595x; 1.0222x over previous
import dataclasses
import functools

import jax
import jax.numpy as jnp
from jax import lax
from jax.experimental import pallas as pl
from jax.experimental.pallas import tpu as pltpu
from jax.experimental.pallas import tpu_sc as plsc

_NW = 32
_L = 16


def _tc_nll_body(SB, B, V, x_ref, t_ref, m_ref, out_ref, tT_ref, mT_ref):
    i = pl.program_id(0)

    @pl.when(i == 0)
    def _():
        out_ref[...] = jnp.zeros_like(out_ref)
        tT_ref[...] = t_ref[...].T
        mT_ref[...] = m_ref[...].T

    x = x_ref[...]
    t = tT_ref[pl.ds(i * SB, SB), :]
    m = mT_ref[pl.ds(i * SB, SB), :]
    col = lax.broadcasted_iota(jnp.int32, (SB, B, V), 2)
    sel = jnp.where(col == t[:, :, None], x, 0.0)
    nll_blk = jnp.sum(jnp.sum(sel, axis=2) * m)
    msk_blk = jnp.sum(m)
    r8 = lax.broadcasted_iota(jnp.int32, (8, 128), 0)
    c128 = lax.broadcasted_iota(jnp.int32, (8, 128), 1)
    out_ref[...] += jnp.where((r8 == 0) & (c128 == 0), nll_blk, 0.0) \
        + jnp.where((r8 == 0) & (c128 == 1), msk_blk, 0.0)


def _sc_match_body(N, MW, G, CH2, CHR,
                   pk_hbm, out_hbm,
                   mt_v, mm_v, idx2_v, val2_v, acc_v, cnt_v, res_v, sem):
    wid = lax.axis_index("s") * 2 + lax.axis_index("c")
    iota = lax.iota(jnp.int32, _L)
    zero = jnp.zeros((_L,), jnp.float32)
    izero = jnp.zeros((_L,), jnp.int32)
    NG = N * G

    b2 = wid * CH2
    pltpu.sync_copy(pk_hbm.at[pl.ds(b2, CH2)], mt_v)
    pltpu.sync_copy(pk_hbm.at[pl.ds(NG + b2, CH2)], mm_v.at[pl.ds(0, CH2)])
    for c in range(CH2 // _L, (G * CHR) // _L):
        mm_v[pl.ds(c * _L, _L)] = izero

    @pl.loop(0, CH2, step=_L)
    def _(c):
        j = b2 + c + iota
        row = jnp.minimum(lax.shift_right_logical(j, 2), N - 1)
        mt = mt_v[pl.ds(c, _L)]
        idx2_v[pl.ds(c, _L)] = 2 * NG + jnp.maximum((mt - 1) * N + row, 0)

    splits = list(range(0, CH2, 128)) + [CH2]
    cps = []
    for lo, hi in zip(splits[:-1], splits[1:]):
        cps.append(pltpu.async_copy(
            pk_hbm.at[idx2_v.at[pl.ds(lo, hi - lo)]],
            val2_v.at[pl.ds(lo, hi - lo)], sem))
    for cp in cps:
        cp.wait()

    acc_v[...] = zero
    cnt_v[...] = zero

    @pl.loop(0, CH2, step=_L)
    def _(c):
        mt = mt_v[pl.ds(c, _L)]
        mm = plsc.bitcast(mm_v[pl.ds(c, _L)], jnp.float32)
        val = plsc.bitcast(val2_v[pl.ds(c, _L)], jnp.float32)
        term = val * jnp.where(mt == 0, 0.0, mm)
        acc_v[...] = acc_v[...] + term

    @pl.loop(0, CHR * G, step=_L * G)
    def _(base):
        rs = zero
        for g in range(G):
            rs = rs + plsc.bitcast(
                plsc.load_gather(mm_v, [iota * G + (base + g)]), jnp.float32)
        cnt_v[...] = cnt_v[...] + jnp.where(rs != 0.0, 1.0, 0.0)

    s2 = jnp.sum(acc_v[...])
    sc = jnp.sum(cnt_v[...])
    res_v[...] = jnp.where(iota == 0, -s2, 0.0) + jnp.where(iota == 1, sc, 0.0)
    pltpu.sync_copy(res_v, out_hbm.at[wid])


def kernel(input, target, mask, match_input, match_target, match_mask):
    B, S, V = input.shape
    MW = match_input.shape[2]
    G = match_target.shape[2]
    N = B * S

    f32 = jnp.float32
    i32 = jnp.int32
    xt = jnp.transpose(input, (1, 0, 2))
    SB = 10

    nll = pl.pallas_call(
        functools.partial(_tc_nll_body, SB, B, V),
        grid=(S // SB,),
        in_specs=[
            pl.BlockSpec((SB, B, V), lambda i: (i, 0, 0)),
            pl.BlockSpec((B, S), lambda i: (0, 0)),
            pl.BlockSpec((B, S), lambda i: (0, 0)),
        ],
        out_specs=pl.BlockSpec((8, 128), lambda i: (0, 0)),
        out_shape=jax.ShapeDtypeStruct((8, 128), f32),
        scratch_shapes=[
            pltpu.VMEM((S, B), i32),
            pltpu.VMEM((S, B), f32),
        ],
    )(xt, target.astype(i32), mask.astype(f32))

    CH2 = N * G // _NW
    CHR = N // _NW
    CHR_PAD = (CHR + _L - 1) // _L * _L

    mif = jnp.transpose(match_input, (2, 0, 1)).reshape(-1)
    pk = jnp.concatenate([
        match_target.reshape(-1).astype(i32),
        lax.bitcast_convert_type(match_mask.reshape(-1).astype(f32), i32),
        lax.bitcast_convert_type(mif, i32),
    ])

    mesh = plsc.VectorSubcoreMesh(core_axis_name="c", subcore_axis_name="s")
    body = functools.partial(_sc_match_body, N, MW, G, CH2, CHR_PAD)
    cp = pltpu.CompilerParams()
    if "needs_layout_passes" in pltpu.CompilerParams.__dataclass_fields__:
        cp = dataclasses.replace(cp, needs_layout_passes=False)
    out = pl.kernel(
        body,
        out_type=jax.ShapeDtypeStruct((_NW, _L), f32),
        mesh=mesh,
        compiler_params=cp,
        scratch_types=[
            pltpu.VMEM((CH2,), i32),
            pltpu.VMEM((G * CHR_PAD,), i32),
            pltpu.VMEM((CH2,), i32),
            pltpu.VMEM((CH2,), i32),
            pltpu.VMEM((_L,), f32),
            pltpu.VMEM((_L,), f32),
            pltpu.VMEM((_L,), f32),
            pltpu.SemaphoreType.DMA,
        ],
    )(pk)

    p = out.sum(axis=0)
    return (-nll[0, 0] / nll[0, 1], p[0] / p[1])
